# Initial kernel scaffold; baseline (speedup 1.0000x reference)
import jax
import jax.numpy as jnp
from jax.experimental import pallas as pl


def kernel(x, edge_index, W_l1, W_r1, b1, W_l2, W_r2, b2):
    raise NotImplementedError("write your pallas kernel here")


# speedup vs baseline: 4.0298x; 4.0298x over previous
import functools

import jax
import jax.numpy as jnp
from jax import lax
from jax.experimental import pallas as pl
from jax.experimental.pallas import tpu as pltpu
from jax.experimental.pallas import tpu_sc as plsc

NW = 32
CHUNK = 128


def _zero_rows(ref, nrows, width16):
    zero16 = jnp.zeros((16,), jnp.float32)

    def zrow(i, _):
        for c in range(width16):
            ref[i, pl.ds(c * 16, 16)] = zero16
        return 0
    lax.fori_loop(0, nrows, zrow, 0)


def _unpack_chunk(pidx, j, sidx, didx, b):
    for c in range(8):
        w = pidx[j, pl.ds(c * 16, 16)]
        if sidx is not None:
            sidx[b, pl.ds(c * 16, 16)] = jnp.bitwise_and(w, 0xFFFF)
        didx[b, pl.ds(c * 16, 16)] = lax.shift_right_logical(w, 16)


def _make_sc_deg(n_pad, nchunk):
    rpt = n_pad // 16
    nz, rem = divmod(rpt, CHUNK)
    mesh = plsc.VectorSubcoreMesh(core_axis_name="c", subcore_axis_name="s")
    scratch = [
        pltpu.VMEM((nchunk, CHUNK), jnp.int32),
        pltpu.VMEM((2, CHUNK), jnp.int32),
        pltpu.VMEM((CHUNK, 128), jnp.float32),
        pltpu.VMEM_SHARED((n_pad, 128), jnp.float32),
    ]

    def body(packed, deg_out, pidx, didx, ones_v, deg_sh):
        c = lax.axis_index("c")
        s = lax.axis_index("s")
        wid = s * 2 + c
        base = s * rpt
        pltpu.sync_copy(packed.at[wid], pidx)

        _zero_rows(ones_v, CHUNK, 8)

        def zcopy(i, _):
            pltpu.sync_copy(ones_v, deg_sh.at[pl.ds(base + i * CHUNK, CHUNK)])
            return 0
        lax.fori_loop(0, nz, zcopy, 0)
        if rem:
            pltpu.sync_copy(ones_v.at[pl.ds(0, rem)],
                            deg_sh.at[pl.ds(base + nz * CHUNK, rem)])

        one16 = jnp.ones((16,), jnp.float32)

        def onerow(i, _):
            for cb in range(8):
                ones_v[i, pl.ds(cb * 16, 16)] = one16
            return 0
        lax.fori_loop(0, CHUNK, onerow, 0)

        plsc.subcore_barrier()

        def chunk(j, _):
            _unpack_chunk(pidx, j, None, didx, 0)
            pltpu.sync_copy(ones_v, deg_sh.at[didx.at[0]], add=True)
            return 0
        lax.fori_loop(0, nchunk, chunk, 0)

        plsc.subcore_barrier()
        pltpu.sync_copy(deg_sh.at[pl.ds(base, rpt)],
                        deg_out.at[c, pl.ds(base, rpt)])

    return functools.partial(
        pl.kernel,
        out_type=[jax.ShapeDtypeStruct((2, n_pad, 128), jnp.float32)],
        mesh=mesh, scratch_types=scratch,
    )(body)


def _make_sc_agg(n_pad, nchunk):
    rpt = n_pad // 16
    nz, rem = divmod(rpt, CHUNK)
    assert nchunk % 2 == 0
    mesh = plsc.VectorSubcoreMesh(core_axis_name="c", subcore_axis_name="s")
    scratch = [
        pltpu.VMEM((nchunk, CHUNK), jnp.int32),
        pltpu.VMEM((2, CHUNK), jnp.int32),
        pltpu.VMEM((2, CHUNK), jnp.int32),
        pltpu.VMEM((CHUNK, 128), jnp.float32),
        pltpu.VMEM((CHUNK, 128), jnp.float32),
        pltpu.VMEM_SHARED((n_pad, 128), jnp.float32),
        pltpu.SemaphoreType.DMA,
        pltpu.SemaphoreType.DMA,
    ]

    def body(table, packed, acc_out, pidx, sidx, didx, rows0, rows1, acc_sh,
             sem0, sem1):
        c = lax.axis_index("c")
        s = lax.axis_index("s")
        wid = s * 2 + c
        base = s * rpt
        pltpu.sync_copy(packed.at[wid], pidx)

        _zero_rows(rows0, CHUNK, 8)

        def zcopy(i, _):
            pltpu.sync_copy(rows0, acc_sh.at[pl.ds(base + i * CHUNK, CHUNK)])
            return 0
        lax.fori_loop(0, nz, zcopy, 0)
        if rem:
            pltpu.sync_copy(rows0.at[pl.ds(0, rem)],
                            acc_sh.at[pl.ds(base + nz * CHUNK, rem)])

        _unpack_chunk(pidx, 0, sidx, didx, 0)
        _unpack_chunk(pidx, 1, sidx, didx, 1)
        pltpu.async_copy(table.at[sidx.at[0]], rows0, sem0)
        pltpu.async_copy(table.at[sidx.at[1]], rows1, sem1)

        plsc.subcore_barrier()

        def group(g, _):
            for b, rows, sem in ((0, rows0, sem0), (1, rows1, sem1)):
                j = 2 * g + b
                pltpu.make_async_copy(table.at[sidx.at[b]], rows, sem).wait()
                pltpu.sync_copy(rows, acc_sh.at[didx.at[b]], add=True)

                @pl.when(j + 2 < nchunk)
                def _():
                    _unpack_chunk(pidx, j + 2, sidx, didx, b)
                    pltpu.async_copy(table.at[sidx.at[b]], rows, sem)
            return 0
        lax.fori_loop(0, nchunk // 2, group, 0)

        plsc.subcore_barrier()
        pltpu.sync_copy(acc_sh.at[pl.ds(base, rpt)],
                        acc_out.at[c, pl.ds(base, rpt)])

    return functools.partial(
        pl.kernel,
        out_type=[jax.ShapeDtypeStruct((2, n_pad, 128), jnp.float32)],
        mesh=mesh, scratch_types=scratch,
    )(body)


def _tc_layer(acc, d0, d1, x, W_l, W_r, b, apply_relu):
    n, dim = x.shape
    blk = 1024
    grid = (pl.cdiv(n, blk),)

    def tc_body(a_ref, d0_ref, d1_ref, x_ref, wl_ref, wr_ref, b_ref, o_ref):
        deg = d0_ref[...] + d1_ref[...]
        recip = 1.0 / jnp.maximum(deg, 1.0)
        mean = (a_ref[0] + a_ref[1]) * recip
        y = jnp.dot(mean, wl_ref[...], preferred_element_type=jnp.float32)
        y += jnp.dot(x_ref[...], wr_ref[...], preferred_element_type=jnp.float32)
        y += b_ref[...]
        if apply_relu:
            y = jnp.maximum(y, 0.0)
        o_ref[...] = y

    return pl.pallas_call(
        tc_body,
        grid=grid,
        in_specs=[
            pl.BlockSpec((2, blk, 128), lambda i: (0, i, 0)),
            pl.BlockSpec((blk, 1), lambda i: (i, 0)),
            pl.BlockSpec((blk, 1), lambda i: (i, 0)),
            pl.BlockSpec((blk, 128), lambda i: (i, 0)),
            pl.BlockSpec((128, 128), lambda i: (0, 0)),
            pl.BlockSpec((128, 128), lambda i: (0, 0)),
            pl.BlockSpec((1, 128), lambda i: (0, 0)),
        ],
        out_specs=pl.BlockSpec((blk, 128), lambda i: (i, 0)),
        out_shape=jax.ShapeDtypeStruct((n, dim), jnp.float32),
    )(acc, d0, d1, x, W_l, W_r, b.reshape(1, 128))


def kernel(x, edge_index, W_l1, W_r1, b1, W_l2, W_r2, b2):
    n = x.shape[0]
    e = edge_index.shape[1]
    n_pad = -(-n // 128) * 128
    ept = -(-e // (NW * CHUNK)) * CHUNK
    if ept % (2 * CHUNK) != 0:
        ept += CHUNK
    nchunk = ept // CHUNK
    e_pad = NW * ept

    src = edge_index[0].astype(jnp.int32)
    dst = edge_index[1].astype(jnp.int32)
    pad = e_pad - e
    src_p = jnp.concatenate([src, jnp.zeros((pad,), jnp.int32)])
    dst_p = jnp.concatenate([dst, jnp.full((pad,), n_pad - 1, jnp.int32)])
    packed = jnp.bitwise_or(jnp.left_shift(dst_p, 16), src_p)
    packed = packed.reshape(NW, nchunk, CHUNK)

    (deg,) = _make_sc_deg(n_pad, nchunk)(packed)
    d0 = deg[0, :n, 0:1]
    d1 = deg[1, :n, 0:1]
    (agg1,) = _make_sc_agg(n_pad, nchunk)(x, packed)
    h = _tc_layer(agg1, d0, d1, x, W_l1, W_r1, b1, True)
    (agg2,) = _make_sc_agg(n_pad, nchunk)(h, packed)
    out = _tc_layer(agg2, d0, d1, h, W_l2, W_r2, b2, False)
    return out
